# Initial kernel scaffold; baseline (speedup 1.0000x reference)
import jax
import jax.numpy as jnp
from jax.experimental import pallas as pl


def kernel(data, table):
    raise NotImplementedError("write your pallas kernel here")


# speedup vs baseline: 1.8434x; 1.8434x over previous
import functools

import jax
import jax.numpy as jnp
from jax import lax
from jax.experimental import pallas as pl
from jax.experimental.pallas import tpu as pltpu
from jax.experimental.pallas import tpu_sc as plsc

D_MODEL = 64


def _gather_kernel(n_total, n_workers, chunk, num_cores):
    n_per_w = n_total // n_workers
    n_chunks = n_per_w // chunk

    mesh = plsc.VectorSubcoreMesh(core_axis_name="c", subcore_axis_name="s")

    @functools.partial(
        pl.kernel,
        mesh=mesh,
        out_type=jax.ShapeDtypeStruct((n_total, D_MODEL), jnp.float32),
        scratch_types=[
            pltpu.VMEM((chunk,), jnp.int32),
            pltpu.VMEM((chunk, D_MODEL), jnp.float32),
            pltpu.SemaphoreType.DMA,
        ],
        compiler_params=pltpu.CompilerParams(use_tc_tiling_on_sc=False),
    )
    def k(idx_hbm, table_hbm, out_hbm, idx_v, rows_v, sem):
        wid = lax.axis_index("s") * num_cores + lax.axis_index("c")
        base = wid * n_per_w

        def body(i, carry):
            off = base + i * chunk
            pltpu.sync_copy(idx_hbm.at[pl.ds(off, chunk)], idx_v)
            pltpu.async_copy(table_hbm.at[idx_v], rows_v, sem).wait()
            pltpu.sync_copy(rows_v, out_hbm.at[pl.ds(off, chunk)])
            return carry

        lax.fori_loop(0, n_chunks, body, 0)

    return k


def kernel(data, table):
    batch, hist = data.shape
    n_total = batch * hist
    info = plsc.get_sparse_core_info()
    n_workers = info.num_cores * info.num_subcores
    chunk = 1024

    flat_idx = data.reshape(n_total)
    k = _gather_kernel(n_total, n_workers, chunk, info.num_cores)
    out = k(flat_idx, table)
    return out.reshape(batch, hist, D_MODEL)


# speedup vs baseline: 1.8718x; 1.0154x over previous
import functools

import jax
import jax.numpy as jnp
from jax import lax
from jax.experimental import pallas as pl
from jax.experimental.pallas import tpu as pltpu
from jax.experimental.pallas import tpu_sc as plsc

D_MODEL = 64


def _gather_kernel(n_total, n_workers, chunk, num_cores):
    n_per_w = n_total // n_workers
    n_chunks = n_per_w // chunk
    assert n_chunks * chunk == n_per_w and n_chunks % 2 == 0

    mesh = plsc.VectorSubcoreMesh(core_axis_name="c", subcore_axis_name="s")

    @functools.partial(
        pl.kernel,
        mesh=mesh,
        out_type=jax.ShapeDtypeStruct((n_total, D_MODEL), jnp.float32),
        scratch_types=[
            pltpu.VMEM((n_per_w,), jnp.int32),
            pltpu.VMEM((2, chunk, D_MODEL), jnp.float32),
            pltpu.SemaphoreType.DMA,
            pltpu.SemaphoreType.DMA,
        ],
        compiler_params=pltpu.CompilerParams(use_tc_tiling_on_sc=False),
    )
    def k(idx_hbm, table_hbm, out_hbm, idx_v, rows_v, sem_g, sem_o):
        wid = lax.axis_index("s") * num_cores + lax.axis_index("c")
        base = wid * n_per_w

        pltpu.sync_copy(idx_hbm.at[pl.ds(base, n_per_w)], idx_v)
        pltpu.async_copy(table_hbm.at[idx_v.at[pl.ds(0, chunk)]],
                         rows_v.at[0], sem_g)

        def outer(i2, carry):
            for b in range(2):
                i = i2 * 2 + b
                slot, nxt = b, 1 - b

                @pl.when(i >= 1)
                def _():
                    pltpu.make_async_copy(
                        rows_v.at[nxt],
                        out_hbm.at[pl.ds(base, chunk)], sem_o).wait()

                @pl.when(i + 1 < n_chunks)
                def _():
                    pltpu.async_copy(
                        table_hbm.at[idx_v.at[pl.ds((i + 1) * chunk, chunk)]],
                        rows_v.at[nxt], sem_g)

                pltpu.make_async_copy(
                    table_hbm.at[idx_v.at[pl.ds(0, chunk)]],
                    rows_v.at[slot], sem_g).wait()
                pltpu.async_copy(rows_v.at[slot],
                                 out_hbm.at[pl.ds(base + i * chunk, chunk)],
                                 sem_o)
            return carry

        lax.fori_loop(0, n_chunks // 2, outer, 0)
        pltpu.make_async_copy(rows_v.at[1],
                              out_hbm.at[pl.ds(base, chunk)], sem_o).wait()

    return k


def kernel(data, table):
    batch, hist = data.shape
    n_total = batch * hist
    info = plsc.get_sparse_core_info()
    n_workers = info.num_cores * info.num_subcores
    chunk = 640

    flat_idx = data.reshape(n_total)
    k = _gather_kernel(n_total, n_workers, chunk, info.num_cores)
    out = k(flat_idx, table)
    return out.reshape(batch, hist, D_MODEL)


# speedup vs baseline: 1.8778x; 1.0032x over previous
import functools

import jax
import jax.numpy as jnp
from jax import lax
from jax.experimental import pallas as pl
from jax.experimental.pallas import tpu as pltpu
from jax.experimental.pallas import tpu_sc as plsc

D_MODEL = 64


def _gather_kernel(n_total, n_workers, chunk, num_cores, nbuf):
    n_per_w = n_total // n_workers
    n_chunks = n_per_w // chunk
    assert n_chunks * chunk == n_per_w and n_chunks % nbuf == 0
    assert n_chunks >= nbuf

    mesh = plsc.VectorSubcoreMesh(core_axis_name="c", subcore_axis_name="s")

    @functools.partial(
        pl.kernel,
        mesh=mesh,
        out_type=jax.ShapeDtypeStruct((n_total, D_MODEL), jnp.float32),
        scratch_types=[
            pltpu.VMEM((n_per_w,), jnp.int32),
            pltpu.VMEM((nbuf, chunk, D_MODEL), jnp.float32),
            pltpu.SemaphoreType.DMA,
            pltpu.SemaphoreType.DMA,
        ],
        compiler_params=pltpu.CompilerParams(use_tc_tiling_on_sc=False),
    )
    def k(idx_hbm, table_hbm, out_hbm, idx_v, rows_v, sem_g, sem_o):
        wid = lax.axis_index("s") * num_cores + lax.axis_index("c")
        base = wid * n_per_w

        pltpu.sync_copy(idx_hbm.at[pl.ds(base, n_per_w)], idx_v)
        for j in range(nbuf - 1):
            pltpu.async_copy(table_hbm.at[idx_v.at[pl.ds(j * chunk, chunk)]],
                             rows_v.at[j], sem_g)

        def outer(i2, carry):
            for b in range(nbuf):
                i = i2 * nbuf + b
                slot = b
                ahead = (b + nbuf - 1) % nbuf

                @pl.when(i >= 1)
                def _():
                    pltpu.make_async_copy(
                        rows_v.at[ahead],
                        out_hbm.at[pl.ds(base, chunk)], sem_o).wait()

                @pl.when(i + nbuf - 1 < n_chunks)
                def _():
                    pltpu.async_copy(
                        table_hbm.at[
                            idx_v.at[pl.ds((i + nbuf - 1) * chunk, chunk)]],
                        rows_v.at[ahead], sem_g)

                pltpu.make_async_copy(
                    table_hbm.at[idx_v.at[pl.ds(0, chunk)]],
                    rows_v.at[slot], sem_g).wait()
                pltpu.async_copy(rows_v.at[slot],
                                 out_hbm.at[pl.ds(base + i * chunk, chunk)],
                                 sem_o)
            return carry

        lax.fori_loop(0, n_chunks // nbuf, outer, 0)
        pltpu.make_async_copy(rows_v.at[(n_chunks - 1) % nbuf],
                              out_hbm.at[pl.ds(base, chunk)], sem_o).wait()

    return k


def kernel(data, table):
    batch, hist = data.shape
    n_total = batch * hist
    info = plsc.get_sparse_core_info()
    n_workers = info.num_cores * info.num_subcores
    chunk = 256
    nbuf = 4

    flat_idx = data.reshape(n_total)
    k = _gather_kernel(n_total, n_workers, chunk, info.num_cores, nbuf)
    out = k(flat_idx, table)
    return out.reshape(batch, hist, D_MODEL)
